# baseline (speedup 1.0000x reference)
import functools

import jax
import jax.numpy as jnp
from jax import lax
from jax.experimental import pallas as pl
from jax.experimental.pallas import tpu as pltpu
from jax.experimental.pallas import tpu_sc as plsc

F32 = jnp.float32
_NC = 2
_NS = 16
_NW = _NC * _NS
_CH = 128

_SC_PARAMS = pltpu.CompilerParams(use_tc_tiling_on_sc=False)


def _rnd(n, m):
    return ((n + m - 1) // m) * m


def _bm(m, cap=2048):
    b = cap
    while m % b:
        b //= 2
    return b


@functools.lru_cache(maxsize=None)
def _make_gather(npad, d, epad):
    b = epad // _NW
    nch = b // _CH
    mesh = plsc.VectorSubcoreMesh(core_axis_name="c", subcore_axis_name="s")

    @functools.partial(
        pl.kernel,
        out_type=jax.ShapeDtypeStruct((epad, d), F32),
        mesh=mesh,
        compiler_params=_SC_PARAMS,
        scratch_types=[
            pltpu.VMEM((b,), jnp.int32),
            pltpu.VMEM((b, d), F32),
            pltpu.SemaphoreType.DMA,
        ],
    )
    def gather_k(table_hbm, idx_hbm, out_hbm, idx_v, rows_v, sem):
        wid = lax.axis_index("s") * _NC + lax.axis_index("c")
        pltpu.sync_copy(idx_hbm.at[pl.ds(wid * b, b)], idx_v)
        cps = []
        for j in range(nch):
            cps.append(pltpu.async_copy(
                table_hbm.at[idx_v.at[pl.ds(j * _CH, _CH)]],
                rows_v.at[pl.ds(j * _CH, _CH)], sem))
        for cp in cps:
            cp.wait()
        pltpu.sync_copy(rows_v, out_hbm.at[pl.ds(wid * b, b)])

    return gather_k


@functools.lru_cache(maxsize=None)
def _make_scatter(npad, d, epad, with_cnt=False):
    b = epad // _NW
    nch = b // _CH
    zr = npad // _NS
    mesh = plsc.VectorSubcoreMesh(core_axis_name="c", subcore_axis_name="s")

    out_type = [jax.ShapeDtypeStruct((_NC * npad, d), F32)]
    scratch = [
        pltpu.VMEM((b, d), F32),
        pltpu.VMEM((nch, _CH), jnp.int32),
        pltpu.VMEM_SHARED((npad, d), F32),
        pltpu.SemaphoreType.DMA,
    ]
    if with_cnt:
        out_type.append(jax.ShapeDtypeStruct((_NC * npad, 16), F32))
        scratch += [pltpu.VMEM((_CH, 16), F32),
                    pltpu.VMEM_SHARED((npad, 16), F32)]

    def scatter_body(vals_hbm, idx_hbm, zeros_hbm, *rest):
        if with_cnt:
            (zeros_c_hbm, ones_hbm, out_hbm, outc_hbm,
             vals_v, idx_v, acc_sh, sem, ones_v, accc_sh) = rest
        else:
            out_hbm, vals_v, idx_v, acc_sh, sem = rest
        c = lax.axis_index("c")
        s = lax.axis_index("s")
        wid = s * _NC + c
        pltpu.sync_copy(zeros_hbm.at[pl.ds(s * zr, zr)],
                        acc_sh.at[pl.ds(s * zr, zr)])
        if with_cnt:
            pltpu.sync_copy(zeros_c_hbm.at[pl.ds(s * zr, zr)],
                            accc_sh.at[pl.ds(s * zr, zr)])
            pltpu.sync_copy(ones_hbm, ones_v)
        plsc.subcore_barrier()
        pltpu.sync_copy(vals_hbm.at[pl.ds(wid * b, b)], vals_v)
        pltpu.sync_copy(idx_hbm.at[wid], idx_v)
        for j in range(nch):
            pltpu.sync_copy(vals_v.at[pl.ds(j * _CH, _CH)],
                            acc_sh.at[idx_v.at[j]], add=True)
            if with_cnt:
                pltpu.sync_copy(ones_v, accc_sh.at[idx_v.at[j]], add=True)
        plsc.subcore_barrier()
        pltpu.sync_copy(acc_sh.at[pl.ds(s * zr, zr)],
                        out_hbm.at[pl.ds(c * npad + s * zr, zr)])
        if with_cnt:
            pltpu.sync_copy(accc_sh.at[pl.ds(s * zr, zr)],
                            outc_hbm.at[pl.ds(c * npad + s * zr, zr)])

    return pl.kernel(
        scatter_body,
        out_type=out_type if with_cnt else out_type[0],
        mesh=mesh,
        compiler_params=_SC_PARAMS,
        scratch_types=scratch,
    )


def _mm(x, wt, bias, act=None):
    m, k = x.shape
    n = wt.shape[1]
    bm = _bm(m)

    def body(x_ref, w_ref, b_ref, o_ref):
        y = jnp.dot(x_ref[...], w_ref[...], preferred_element_type=F32) + b_ref[...]
        if act == 'relu':
            y = jnp.maximum(y, 0.0)
        o_ref[...] = y

    return pl.pallas_call(
        body,
        grid=(m // bm,),
        in_specs=[pl.BlockSpec((bm, k), lambda i: (i, 0)),
                  pl.BlockSpec((k, n), lambda i: (0, 0)),
                  pl.BlockSpec((1, n), lambda i: (0, 0))],
        out_specs=pl.BlockSpec((bm, n), lambda i: (i, 0)),
        out_shape=jax.ShapeDtypeStruct((m, n), F32),
    )(x, wt, bias)


def _msg(hsrc, g1, a_mat, r_mat, s_mat, h):
    e = hsrc.shape[0]
    n = a_mat.shape[1]
    bm = _bm(e)
    bf16 = jnp.bfloat16

    def body(h_ref, g_ref, a_ref, r_ref, s_ref, o_ref):
        p = jnp.dot(h_ref[...].astype(bf16), a_ref[...],
                    preferred_element_type=F32)
        t = jnp.dot(g_ref[...].astype(bf16), r_ref[...],
                    preferred_element_type=F32)
        q = (p * t).astype(bf16)
        o_ref[...] = jnp.dot(q, s_ref[...], preferred_element_type=F32)

    return pl.pallas_call(
        body,
        grid=(e // bm,),
        in_specs=[pl.BlockSpec((bm, hsrc.shape[1]), lambda i: (i, 0)),
                  pl.BlockSpec((bm, g1.shape[1]), lambda i: (i, 0)),
                  pl.BlockSpec((a_mat.shape[0], n), lambda i: (0, 0)),
                  pl.BlockSpec((r_mat.shape[0], n), lambda i: (0, 0)),
                  pl.BlockSpec((n, h), lambda i: (0, 0))],
        out_specs=pl.BlockSpec((bm, h), lambda i: (i, 0)),
        out_shape=jax.ShapeDtypeStruct((e, h), F32),
    )(hsrc, g1, a_mat, r_mat, s_mat)


def _gru(acc, cnt, hprev, wit, wht, bi, bh):
    npad, d = hprev.shape
    bm = _bm(npad)
    nb = npad // bm

    def body(a0_ref, a1_ref, c0_ref, c1_ref, h_ref, wi_ref, wh_ref,
             bi_ref, bh_ref, o_ref):
        s = a0_ref[...] + a1_ref[...]
        c = jnp.maximum(c0_ref[:, :1] + c1_ref[:, :1], 1.0)
        m = jnp.maximum(s / c, 0.0)
        hh = h_ref[...]
        gi = jnp.dot(m, wi_ref[...], preferred_element_type=F32) + bi_ref[...]
        gh = jnp.dot(hh, wh_ref[...], preferred_element_type=F32) + bh_ref[...]
        r = jax.nn.sigmoid(gi[:, :d] + gh[:, :d])
        z = jax.nn.sigmoid(gi[:, d:2 * d] + gh[:, d:2 * d])
        nn = jnp.tanh(gi[:, 2 * d:] + r * gh[:, 2 * d:])
        o_ref[...] = (1.0 - z) * nn + z * hh

    return pl.pallas_call(
        body,
        grid=(nb,),
        in_specs=[pl.BlockSpec((bm, d), lambda i: (i, 0)),
                  pl.BlockSpec((bm, d), lambda i, nb=nb: (i + nb, 0)),
                  pl.BlockSpec((bm, 16), lambda i: (i, 0)),
                  pl.BlockSpec((bm, 16), lambda i, nb=nb: (i + nb, 0)),
                  pl.BlockSpec((bm, d), lambda i: (i, 0)),
                  pl.BlockSpec((d, 3 * d), lambda i: (0, 0)),
                  pl.BlockSpec((d, 3 * d), lambda i: (0, 0)),
                  pl.BlockSpec((1, 3 * d), lambda i: (0, 0)),
                  pl.BlockSpec((1, 3 * d), lambda i: (0, 0))],
        out_specs=pl.BlockSpec((bm, d), lambda i: (i, 0)),
        out_shape=jax.ShapeDtypeStruct((npad, d), F32),
    )(acc, acc, cnt, cnt, hprev, wit, wht, bi, bh)


def _frag_assemble(ff, wt, bias, acc, cnt):
    npad = ff.shape[0]
    k = ff.shape[1]
    d = wt.shape[1]
    bm = _bm(npad)
    nb = npad // bm

    def body(f_ref, w_ref, b_ref, a0_ref, a1_ref, c0_ref, c1_ref, o_ref):
        emb = jnp.dot(f_ref[...], w_ref[...], preferred_element_type=F32) + b_ref[...]
        s = a0_ref[...] + a1_ref[...]
        c = jnp.maximum(c0_ref[:, :1] + c1_ref[:, :1], 1.0)
        o_ref[...] = jnp.concatenate([emb, s / c], axis=1)

    return pl.pallas_call(
        body,
        grid=(nb,),
        in_specs=[pl.BlockSpec((bm, k), lambda i: (i, 0)),
                  pl.BlockSpec((k, d), lambda i: (0, 0)),
                  pl.BlockSpec((1, d), lambda i: (0, 0)),
                  pl.BlockSpec((bm, d), lambda i: (i, 0)),
                  pl.BlockSpec((bm, d), lambda i, nb=nb: (i + nb, 0)),
                  pl.BlockSpec((bm, 16), lambda i: (i, 0)),
                  pl.BlockSpec((bm, 16), lambda i, nb=nb: (i + nb, 0))],
        out_specs=pl.BlockSpec((bm, 2 * d), lambda i: (i, 0)),
        out_shape=jax.ShapeDtypeStruct((npad, 2 * d), F32),
    )(ff, wt, bias, acc, acc, cnt, cnt)


def _final(acc, cnt, wt, bias, eps, nb_real, latent):
    npad = acc.shape[0] // 2
    d = acc.shape[1]

    def body(a0_ref, a1_ref, c0_ref, c1_ref, w_ref, b_ref, e_ref,
             z_ref, mu_ref, lv_ref):
        s = a0_ref[...] + a1_ref[...]
        c = jnp.maximum(c0_ref[:, :1] + c1_ref[:, :1], 1.0)
        hm = (s / c)[:nb_real]
        x = jnp.dot(hm, w_ref[...], preferred_element_type=F32) + b_ref[...]
        mu = x[:, :latent]
        lv = x[:, latent:]
        std = jnp.exp(0.5 * lv)
        z_ref[...] = mu + e_ref[...] * std
        mu_ref[...] = mu
        lv_ref[...] = lv

    out = jax.ShapeDtypeStruct((nb_real, latent), F32)
    return pl.pallas_call(
        body,
        grid=(1,),
        in_specs=[pl.BlockSpec((npad, d), lambda i: (0, 0)),
                  pl.BlockSpec((npad, d), lambda i: (1, 0)),
                  pl.BlockSpec((npad, 16), lambda i: (0, 0)),
                  pl.BlockSpec((npad, 16), lambda i: (1, 0)),
                  pl.BlockSpec((d, 2 * latent), lambda i: (0, 0)),
                  pl.BlockSpec((1, 2 * latent), lambda i: (0, 0)),
                  pl.BlockSpec((nb_real, latent), lambda i: (0, 0))],
        out_specs=[pl.BlockSpec((nb_real, latent), lambda i: (0, 0))] * 3,
        out_shape=[out, out, out],
    )(acc, acc, cnt, cnt, wt, bias, eps)


def _edge_net_mat(e2w, e2b, h, k):
    a = e2w.reshape(h, h, k).transpose(0, 2, 1).reshape(h, k * h)
    return jnp.concatenate([a, e2b.reshape(h, h)], axis=1)


def _pad_idx(idx, epad, dump):
    return jnp.pad(idx, (0, epad - idx.shape[0]), constant_values=dump)


def _pad_idx3(idx, epad, dump):
    return jnp.pad(idx, (0, epad - idx.shape[0]),
                   constant_values=dump).reshape(_NW, -1, _CH)


def kernel(atom_feat, atom_bond_feat, frag_feat, fbond_feat, atom_edge_index,
           atom_graph_ids, frag_edge_index, frag_graph_ids, eps, params):
    p = params
    na, ea = atom_feat.shape[0], atom_edge_index.shape[1]
    nf, ef = frag_feat.shape[0], frag_edge_index.shape[1]
    nb = eps.shape[0]
    latent = eps.shape[1]
    ha = p['emb_atom_W'].shape[0]
    hf = 2 * p['emb_frag_W'].shape[0]
    ka = p['amp']['e1_W'].shape[0]
    kf = p['fmp']['e1_W'].shape[0]

    nap = _rnd(na + 1, 1024)
    nfp = _rnd(nf + 1, 1024)
    nbp = _rnd(nb + 1, 128)
    eap = _rnd(ea, _NW * _CH)
    efp = _rnd(ef, _NW * _CH)
    iap = _rnd(max(na, nap), _NW * _CH)
    ifp = _rnd(max(nf, nfp), _NW * _CH)

    a_src = _pad_idx(atom_edge_index[0], eap, nap - 1)
    a_dst = _pad_idx3(atom_edge_index[1], eap, nap - 1)
    f_src = _pad_idx(frag_edge_index[0], efp, nfp - 1)
    f_dst = _pad_idx3(frag_edge_index[1], efp, nfp - 1)
    a_gid = _pad_idx3(atom_graph_ids, iap, nfp - 1)
    f_gid = _pad_idx3(frag_graph_ids, ifp, nbp - 1)

    amp, fmp = p['amp'], p['fmp']
    bf16 = jnp.bfloat16
    w_bond = (amp['e1_W'] @ p['emb_bond_W']).T
    b_bond = (p['emb_bond_b'] @ amp['e1_W'].T + amp['e1_b'])[None]
    w_fbond = (fmp['e1_W'] @ p['emb_fbond_W']).T
    b_fbond = (p['emb_fbond_b'] @ fmp['e1_W'].T + fmp['e1_b'])[None]
    w_bond = jnp.pad(w_bond, ((0, 0), (0, 1)))
    b_bond = jnp.concatenate([b_bond, jnp.ones((1, 1), F32)], axis=1)
    w_fbond = jnp.pad(w_fbond, ((0, 0), (0, 1)))
    b_fbond = jnp.concatenate([b_fbond, jnp.ones((1, 1), F32)], axis=1)
    a_mat_a = _edge_net_mat(amp['e2_W'], amp['e2_b'], ha, ka).astype(bf16)
    a_mat_f = _edge_net_mat(fmp['e2_W'], fmp['e2_b'], hf, kf).astype(bf16)
    r_a = jnp.kron(jnp.eye(ka + 1, dtype=F32), jnp.ones((1, ha), F32)).astype(bf16)
    s_a = jnp.tile(jnp.eye(ha, dtype=F32), (ka + 1, 1)).astype(bf16)
    r_f = jnp.kron(jnp.eye(kf + 1, dtype=F32), jnp.ones((1, hf), F32)).astype(bf16)
    s_f = jnp.tile(jnp.eye(hf, dtype=F32), (kf + 1, 1)).astype(bf16)

    zeros_a = jnp.zeros((nap, ha), F32)
    zeros_f32 = jnp.zeros((nfp, ha), F32)
    zeros_f64 = jnp.zeros((nfp, hf), F32)
    zeros_b = jnp.zeros((nbp, hf), F32)
    zeros_ca = jnp.zeros((nap, 16), F32)
    zeros_cf = jnp.zeros((nfp, 16), F32)
    zeros_cb = jnp.zeros((nbp, 16), F32)

    ones16 = jnp.ones((_CH, 16), F32)

    af = jnp.pad(atom_feat, ((0, nap - na), (0, 0)))
    h = _mm(af, p['emb_atom_W'].T, p['emb_atom_b'][None])
    bf = jnp.pad(atom_bond_feat, ((0, eap - ea), (0, 0)))
    g_a = _mm(bf, w_bond, b_bond, act='relu')

    gather_a = _make_gather(nap, ha, eap)
    wit_a, wht_a = amp['gru_Wih'].T, amp['gru_Whh'].T
    bi_a, bh_a = amp['gru_bih'][None], amp['gru_bhh'][None]
    cnt_a = None
    for it in range(2):
        hs = gather_a(h, a_src)
        msg = _msg(hs, g_a, a_mat_a, r_a, s_a, ha)
        if it == 0:
            acc, cnt_a = _make_scatter(nap, ha, eap, True)(
                msg, a_dst, zeros_a, zeros_ca, ones16)
        else:
            acc = _make_scatter(nap, ha, eap)(msg, a_dst, zeros_a)
        h = _gru(acc, cnt_a, h, wit_a, wht_a, bi_a, bh_a)

    h_pad = jnp.pad(h, ((0, iap - nap), (0, 0)))
    acc_af, cnt_af = _make_scatter(nfp, ha, iap, True)(
        h_pad, a_gid, zeros_f32, zeros_cf, ones16)
    ffp = jnp.pad(frag_feat, ((0, nfp - nf), (0, 0)))
    hfr = _frag_assemble(ffp, p['emb_frag_W'].T, p['emb_frag_b'][None],
                         acc_af, cnt_af)

    fbf = jnp.pad(fbond_feat, ((0, efp - ef), (0, 0)))
    g_f = _mm(fbf, w_fbond, b_fbond, act='relu')
    gather_f = _make_gather(nfp, hf, efp)
    wit_f, wht_f = fmp['gru_Wih'].T, fmp['gru_Whh'].T
    bi_f, bh_f = fmp['gru_bih'][None], fmp['gru_bhh'][None]
    cnt_f = None
    for it in range(2):
        hs = gather_f(hfr, f_src)
        msg = _msg(hs, g_f, a_mat_f, r_f, s_f, hf)
        if it == 0:
            acc, cnt_f = _make_scatter(nfp, hf, efp, True)(
                msg, f_dst, zeros_f64, zeros_cf, ones16)
        else:
            acc = _make_scatter(nfp, hf, efp)(msg, f_dst, zeros_f64)
        hfr = _gru(acc, cnt_f, hfr, wit_f, wht_f, bi_f, bh_f)

    hfr_pad = jnp.pad(hfr, ((0, ifp - nfp), (0, 0)))
    acc_fb, cnt_fb = _make_scatter(nbp, hf, ifp, True)(
        hfr_pad, f_gid, zeros_b, zeros_cb, ones16)
    z, mu, lv = _final(acc_fb, cnt_fb, p['enc_W'].T, p['enc_b'][None],
                       eps, nb, latent)
    return (z, mu, lv)


# speedup vs baseline: 1.0788x; 1.0788x over previous
import functools

import jax
import jax.numpy as jnp
from jax import lax
from jax.experimental import pallas as pl
from jax.experimental.pallas import tpu as pltpu
from jax.experimental.pallas import tpu_sc as plsc

F32 = jnp.float32
_NC = 2
_NS = 16
_NW = _NC * _NS
_CH = 128

_SC_PARAMS = pltpu.CompilerParams(use_tc_tiling_on_sc=False)


def _rnd(n, m):
    return ((n + m - 1) // m) * m


def _bm(m, cap=2048):
    b = cap
    while m % b:
        b //= 2
    return b


@functools.lru_cache(maxsize=None)
def _make_gather(npad, d, epad):
    b = epad // _NW
    nch = b // _CH
    mesh = plsc.VectorSubcoreMesh(core_axis_name="c", subcore_axis_name="s")

    @functools.partial(
        pl.kernel,
        out_type=jax.ShapeDtypeStruct((epad, d), F32),
        mesh=mesh,
        compiler_params=_SC_PARAMS,
        scratch_types=[
            pltpu.VMEM((b,), jnp.int32),
            pltpu.VMEM((b, d), F32),
            pltpu.SemaphoreType.DMA,
        ],
    )
    def gather_k(table_hbm, idx_hbm, out_hbm, idx_v, rows_v, sem):
        wid = lax.axis_index("s") * _NC + lax.axis_index("c")
        pltpu.sync_copy(idx_hbm.at[pl.ds(wid * b, b)], idx_v)
        cps = []
        for j in range(nch):
            cps.append(pltpu.async_copy(
                table_hbm.at[idx_v.at[pl.ds(j * _CH, _CH)]],
                rows_v.at[pl.ds(j * _CH, _CH)], sem))
        for cp in cps:
            cp.wait()
        pltpu.sync_copy(rows_v, out_hbm.at[pl.ds(wid * b, b)])

    return gather_k


@functools.lru_cache(maxsize=None)
def _make_scatter(npad, d, epad, with_cnt=False):
    b = epad // _NW
    nch = b // _CH
    zr = npad // _NS
    mesh = plsc.VectorSubcoreMesh(core_axis_name="c", subcore_axis_name="s")

    out_type = [jax.ShapeDtypeStruct((_NC * npad, d), F32)]
    scratch = [
        pltpu.VMEM((b, d), F32),
        pltpu.VMEM((nch, _CH), jnp.int32),
        pltpu.VMEM_SHARED((npad, d), F32),
        pltpu.SemaphoreType.DMA,
    ]
    if with_cnt:
        out_type.append(jax.ShapeDtypeStruct((_NC * npad, 16), F32))
        scratch += [pltpu.VMEM((_CH, 16), F32),
                    pltpu.VMEM_SHARED((npad, 16), F32)]

    def scatter_body(vals_hbm, idx_hbm, zeros_hbm, *rest):
        if with_cnt:
            (zeros_c_hbm, ones_hbm, out_hbm, outc_hbm,
             vals_v, idx_v, acc_sh, sem, ones_v, accc_sh) = rest
        else:
            out_hbm, vals_v, idx_v, acc_sh, sem = rest
        c = lax.axis_index("c")
        s = lax.axis_index("s")
        wid = s * _NC + c
        pltpu.sync_copy(zeros_hbm.at[pl.ds(s * zr, zr)],
                        acc_sh.at[pl.ds(s * zr, zr)])
        if with_cnt:
            pltpu.sync_copy(zeros_c_hbm.at[pl.ds(s * zr, zr)],
                            accc_sh.at[pl.ds(s * zr, zr)])
            pltpu.sync_copy(ones_hbm, ones_v)
        plsc.subcore_barrier()
        pltpu.sync_copy(vals_hbm.at[pl.ds(wid * b, b)], vals_v)
        pltpu.sync_copy(idx_hbm.at[wid], idx_v)
        for j in range(nch):
            pltpu.sync_copy(vals_v.at[pl.ds(j * _CH, _CH)],
                            acc_sh.at[idx_v.at[j]], add=True)
            if with_cnt:
                pltpu.sync_copy(ones_v, accc_sh.at[idx_v.at[j]], add=True)
        plsc.subcore_barrier()
        pltpu.sync_copy(acc_sh.at[pl.ds(s * zr, zr)],
                        out_hbm.at[pl.ds(c * npad + s * zr, zr)])
        if with_cnt:
            pltpu.sync_copy(accc_sh.at[pl.ds(s * zr, zr)],
                            outc_hbm.at[pl.ds(c * npad + s * zr, zr)])

    return pl.kernel(
        scatter_body,
        out_type=out_type if with_cnt else out_type[0],
        mesh=mesh,
        compiler_params=_SC_PARAMS,
        scratch_types=scratch,
    )


def _mm(x, wt, bias, act=None):
    m, k = x.shape
    n = wt.shape[1]
    bm = _bm(m)

    def body(x_ref, w_ref, b_ref, o_ref):
        y = jnp.dot(x_ref[...], w_ref[...], preferred_element_type=F32) + b_ref[...]
        if act == 'relu':
            y = jnp.maximum(y, 0.0)
        o_ref[...] = y

    return pl.pallas_call(
        body,
        grid=(m // bm,),
        in_specs=[pl.BlockSpec((bm, k), lambda i: (i, 0)),
                  pl.BlockSpec((k, n), lambda i: (0, 0)),
                  pl.BlockSpec((1, n), lambda i: (0, 0))],
        out_specs=pl.BlockSpec((bm, n), lambda i: (i, 0)),
        out_shape=jax.ShapeDtypeStruct((m, n), F32),
    )(x, wt, bias)


def _msg(hsrc, g1, a_mat, r_mat, s_mat, h, bm=None):
    e = hsrc.shape[0]
    n = a_mat.shape[1]
    bm = bm or _bm(e)
    bf16 = jnp.bfloat16

    def body(h_ref, g_ref, a_ref, r_ref, s_ref, o_ref):
        p = jnp.dot(h_ref[...].astype(bf16), a_ref[...],
                    preferred_element_type=F32)
        t = jnp.dot(g_ref[...].astype(bf16), r_ref[...],
                    preferred_element_type=F32)
        q = (p * t).astype(bf16)
        o_ref[...] = jnp.dot(q, s_ref[...], preferred_element_type=F32)

    return pl.pallas_call(
        body,
        grid=(e // bm,),
        in_specs=[pl.BlockSpec((bm, hsrc.shape[1]), lambda i: (i, 0)),
                  pl.BlockSpec((bm, g1.shape[1]), lambda i: (i, 0)),
                  pl.BlockSpec((a_mat.shape[0], n), lambda i: (0, 0)),
                  pl.BlockSpec((r_mat.shape[0], n), lambda i: (0, 0)),
                  pl.BlockSpec((n, h), lambda i: (0, 0))],
        out_specs=pl.BlockSpec((bm, h), lambda i: (i, 0)),
        out_shape=jax.ShapeDtypeStruct((e, h), F32),
    )(hsrc, g1, a_mat, r_mat, s_mat)


def _gru(acc, cnt, hprev, wit, wht, bi, bh):
    npad, d = hprev.shape
    bm = _bm(npad)
    nb = npad // bm

    def body(a0_ref, a1_ref, c0_ref, c1_ref, h_ref, wi_ref, wh_ref,
             bi_ref, bh_ref, o_ref):
        s = a0_ref[...] + a1_ref[...]
        c = jnp.maximum(c0_ref[:, :1] + c1_ref[:, :1], 1.0)
        m = jnp.maximum(s / c, 0.0)
        hh = h_ref[...]
        gi = jnp.dot(m, wi_ref[...], preferred_element_type=F32) + bi_ref[...]
        gh = jnp.dot(hh, wh_ref[...], preferred_element_type=F32) + bh_ref[...]
        r = jax.nn.sigmoid(gi[:, :d] + gh[:, :d])
        z = jax.nn.sigmoid(gi[:, d:2 * d] + gh[:, d:2 * d])
        nn = jnp.tanh(gi[:, 2 * d:] + r * gh[:, 2 * d:])
        o_ref[...] = (1.0 - z) * nn + z * hh

    return pl.pallas_call(
        body,
        grid=(nb,),
        in_specs=[pl.BlockSpec((bm, d), lambda i: (i, 0)),
                  pl.BlockSpec((bm, d), lambda i, nb=nb: (i + nb, 0)),
                  pl.BlockSpec((bm, 16), lambda i: (i, 0)),
                  pl.BlockSpec((bm, 16), lambda i, nb=nb: (i + nb, 0)),
                  pl.BlockSpec((bm, d), lambda i: (i, 0)),
                  pl.BlockSpec((d, 3 * d), lambda i: (0, 0)),
                  pl.BlockSpec((d, 3 * d), lambda i: (0, 0)),
                  pl.BlockSpec((1, 3 * d), lambda i: (0, 0)),
                  pl.BlockSpec((1, 3 * d), lambda i: (0, 0))],
        out_specs=pl.BlockSpec((bm, d), lambda i: (i, 0)),
        out_shape=jax.ShapeDtypeStruct((npad, d), F32),
    )(acc, acc, cnt, cnt, hprev, wit, wht, bi, bh)


def _frag_assemble(ff, wt, bias, acc, cnt):
    npad = ff.shape[0]
    k = ff.shape[1]
    d = wt.shape[1]
    bm = _bm(npad)
    nb = npad // bm

    def body(f_ref, w_ref, b_ref, a0_ref, a1_ref, c0_ref, c1_ref, o_ref):
        emb = jnp.dot(f_ref[...], w_ref[...], preferred_element_type=F32) + b_ref[...]
        s = a0_ref[...] + a1_ref[...]
        c = jnp.maximum(c0_ref[:, :1] + c1_ref[:, :1], 1.0)
        o_ref[...] = jnp.concatenate([emb, s / c], axis=1)

    return pl.pallas_call(
        body,
        grid=(nb,),
        in_specs=[pl.BlockSpec((bm, k), lambda i: (i, 0)),
                  pl.BlockSpec((k, d), lambda i: (0, 0)),
                  pl.BlockSpec((1, d), lambda i: (0, 0)),
                  pl.BlockSpec((bm, d), lambda i: (i, 0)),
                  pl.BlockSpec((bm, d), lambda i, nb=nb: (i + nb, 0)),
                  pl.BlockSpec((bm, 16), lambda i: (i, 0)),
                  pl.BlockSpec((bm, 16), lambda i, nb=nb: (i + nb, 0))],
        out_specs=pl.BlockSpec((bm, 2 * d), lambda i: (i, 0)),
        out_shape=jax.ShapeDtypeStruct((npad, 2 * d), F32),
    )(ff, wt, bias, acc, acc, cnt, cnt)


def _final(acc, cnt, wt, bias, eps, nb_real, latent):
    npad = acc.shape[0] // 2
    d = acc.shape[1]

    def body(a0_ref, a1_ref, c0_ref, c1_ref, w_ref, b_ref, e_ref,
             z_ref, mu_ref, lv_ref):
        s = a0_ref[...] + a1_ref[...]
        c = jnp.maximum(c0_ref[:, :1] + c1_ref[:, :1], 1.0)
        hm = (s / c)[:nb_real]
        x = jnp.dot(hm, w_ref[...], preferred_element_type=F32) + b_ref[...]
        mu = x[:, :latent]
        lv = x[:, latent:]
        std = jnp.exp(0.5 * lv)
        z_ref[...] = mu + e_ref[...] * std
        mu_ref[...] = mu
        lv_ref[...] = lv

    out = jax.ShapeDtypeStruct((nb_real, latent), F32)
    return pl.pallas_call(
        body,
        grid=(1,),
        in_specs=[pl.BlockSpec((npad, d), lambda i: (0, 0)),
                  pl.BlockSpec((npad, d), lambda i: (1, 0)),
                  pl.BlockSpec((npad, 16), lambda i: (0, 0)),
                  pl.BlockSpec((npad, 16), lambda i: (1, 0)),
                  pl.BlockSpec((d, 2 * latent), lambda i: (0, 0)),
                  pl.BlockSpec((1, 2 * latent), lambda i: (0, 0)),
                  pl.BlockSpec((nb_real, latent), lambda i: (0, 0))],
        out_specs=[pl.BlockSpec((nb_real, latent), lambda i: (0, 0))] * 3,
        out_shape=[out, out, out],
    )(acc, acc, cnt, cnt, wt, bias, eps)


def _edge_net_mat(e2w, e2b, h, k):
    a = e2w.reshape(h, h, k).transpose(0, 2, 1).reshape(h, k * h)
    return jnp.concatenate([a, e2b.reshape(h, h)], axis=1)


def _pad_idx(idx, epad, dump):
    return jnp.pad(idx, (0, epad - idx.shape[0]), constant_values=dump)


def _pad_idx3(idx, epad, dump):
    return jnp.pad(idx, (0, epad - idx.shape[0]),
                   constant_values=dump).reshape(_NW, -1, _CH)


def kernel(atom_feat, atom_bond_feat, frag_feat, fbond_feat, atom_edge_index,
           atom_graph_ids, frag_edge_index, frag_graph_ids, eps, params):
    p = params
    na, ea = atom_feat.shape[0], atom_edge_index.shape[1]
    nf, ef = frag_feat.shape[0], frag_edge_index.shape[1]
    nb = eps.shape[0]
    latent = eps.shape[1]
    ha = p['emb_atom_W'].shape[0]
    hf = 2 * p['emb_frag_W'].shape[0]
    ka = p['amp']['e1_W'].shape[0]
    kf = p['fmp']['e1_W'].shape[0]

    nap = _rnd(na + 1, 1024)
    nfp = _rnd(nf + 1, 1024)
    nbp = _rnd(nb + 1, 128)
    eap = _rnd(ea, _NW * _CH)
    efp = _rnd(ef, _NW * _CH)
    iap = _rnd(max(na, nap), _NW * _CH)
    ifp = _rnd(max(nf, nfp), _NW * _CH)

    a_src = _pad_idx(atom_edge_index[0], eap, nap - 1)
    a_dst = _pad_idx3(atom_edge_index[1], eap, nap - 1)
    f_src = _pad_idx(frag_edge_index[0], efp, nfp - 1)
    f_dst = _pad_idx3(frag_edge_index[1], efp, nfp - 1)
    a_gid = _pad_idx3(atom_graph_ids, iap, nfp - 1)
    f_gid = _pad_idx3(frag_graph_ids, ifp, nbp - 1)

    amp, fmp = p['amp'], p['fmp']
    bf16 = jnp.bfloat16
    w_bond = (amp['e1_W'] @ p['emb_bond_W']).T
    b_bond = (p['emb_bond_b'] @ amp['e1_W'].T + amp['e1_b'])[None]
    w_fbond = (fmp['e1_W'] @ p['emb_fbond_W']).T
    b_fbond = (p['emb_fbond_b'] @ fmp['e1_W'].T + fmp['e1_b'])[None]
    w_bond = jnp.pad(w_bond, ((0, 0), (0, 1)))
    b_bond = jnp.concatenate([b_bond, jnp.ones((1, 1), F32)], axis=1)
    w_fbond = jnp.pad(w_fbond, ((0, 0), (0, 1)))
    b_fbond = jnp.concatenate([b_fbond, jnp.ones((1, 1), F32)], axis=1)
    eye4 = jnp.eye(4, dtype=F32)
    a_mat_a = jnp.kron(eye4, _edge_net_mat(amp['e2_W'], amp['e2_b'], ha, ka)).astype(bf16)
    a_mat_f = jnp.kron(eye4, _edge_net_mat(fmp['e2_W'], fmp['e2_b'], hf, kf)).astype(bf16)
    r_a = jnp.kron(eye4, jnp.kron(jnp.eye(ka + 1, dtype=F32),
                                  jnp.ones((1, ha), F32))).astype(bf16)
    s_a = jnp.kron(eye4, jnp.tile(jnp.eye(ha, dtype=F32), (ka + 1, 1))).astype(bf16)
    r_f = jnp.kron(eye4, jnp.kron(jnp.eye(kf + 1, dtype=F32),
                                  jnp.ones((1, hf), F32))).astype(bf16)
    s_f = jnp.kron(eye4, jnp.tile(jnp.eye(hf, dtype=F32), (kf + 1, 1))).astype(bf16)
    w_bond4 = jnp.kron(eye4, w_bond)
    b_bond4 = jnp.tile(b_bond, (1, 4))
    w_fbond4 = jnp.kron(eye4, w_fbond)
    b_fbond4 = jnp.tile(b_fbond, (1, 4))

    zeros_a = jnp.zeros((nap, ha), F32)
    zeros_f32 = jnp.zeros((nfp, ha), F32)
    zeros_f64 = jnp.zeros((nfp, hf), F32)
    zeros_b = jnp.zeros((nbp, hf), F32)
    zeros_ca = jnp.zeros((nap, 16), F32)
    zeros_cf = jnp.zeros((nfp, 16), F32)
    zeros_cb = jnp.zeros((nbp, 16), F32)

    cnt_a = _make_scatter(nap, 16, eap)(jnp.ones((eap, 16), F32), a_dst, zeros_ca)
    cnt_f = _make_scatter(nfp, 16, efp)(jnp.ones((efp, 16), F32), f_dst, zeros_cf)
    cnt_af = _make_scatter(nfp, 16, iap)(jnp.ones((iap, 16), F32), a_gid, zeros_cf)
    cnt_fb = _make_scatter(nbp, 16, ifp)(jnp.ones((ifp, 16), F32), f_gid, zeros_cb)

    af = jnp.pad(atom_feat, ((0, nap - na), (0, 0)))
    h = _mm(af, p['emb_atom_W'].T, p['emb_atom_b'][None])
    bf4 = jnp.pad(atom_bond_feat, ((0, eap - ea), (0, 0))).reshape(eap // 4, -1)
    g_a = _mm(bf4, w_bond4, b_bond4, act='relu')

    gather_a = _make_gather(nap, ha, eap)
    wit_a, wht_a = amp['gru_Wih'].T, amp['gru_Whh'].T
    bi_a, bh_a = amp['gru_bih'][None], amp['gru_bhh'][None]
    for it in range(2):
        hs4 = gather_a(h, a_src).reshape(eap // 4, 4 * ha)
        msg4 = _msg(hs4, g_a, a_mat_a, r_a, s_a, 4 * ha, bm=512)
        acc = _make_scatter(nap, ha, eap)(msg4.reshape(eap, ha), a_dst, zeros_a)
        h = _gru(acc, cnt_a, h, wit_a, wht_a, bi_a, bh_a)

    h_pad = jnp.pad(h, ((0, iap - nap), (0, 0)))
    acc_af = _make_scatter(nfp, ha, iap)(h_pad, a_gid, zeros_f32)
    ffp = jnp.pad(frag_feat, ((0, nfp - nf), (0, 0)))
    hfr = _frag_assemble(ffp, p['emb_frag_W'].T, p['emb_frag_b'][None],
                         acc_af, cnt_af)

    fbf4 = jnp.pad(fbond_feat, ((0, efp - ef), (0, 0))).reshape(efp // 4, -1)
    g_f = _mm(fbf4, w_fbond4, b_fbond4, act='relu')
    gather_f = _make_gather(nfp, hf, efp)
    wit_f, wht_f = fmp['gru_Wih'].T, fmp['gru_Whh'].T
    bi_f, bh_f = fmp['gru_bih'][None], fmp['gru_bhh'][None]
    for it in range(2):
        hs4 = gather_f(hfr, f_src).reshape(efp // 4, 4 * hf)
        msg4 = _msg(hs4, g_f, a_mat_f, r_f, s_f, 4 * hf, bm=512)
        acc = _make_scatter(nfp, hf, efp)(msg4.reshape(efp, hf), f_dst, zeros_f64)
        hfr = _gru(acc, cnt_f, hfr, wit_f, wht_f, bi_f, bh_f)

    hfr_pad = jnp.pad(hfr, ((0, ifp - nfp), (0, 0)))
    acc_fb = _make_scatter(nbp, hf, ifp)(hfr_pad, f_gid, zeros_b)
    z, mu, lv = _final(acc_fb, cnt_fb, p['enc_W'].T, p['enc_b'][None],
                       eps, nb, latent)
    return (z, mu, lv)


# speedup vs baseline: 1.3419x; 1.2439x over previous
import functools

import jax
import jax.numpy as jnp
from jax import lax
from jax.experimental import pallas as pl
from jax.experimental.pallas import tpu as pltpu
from jax.experimental.pallas import tpu_sc as plsc

F32 = jnp.float32
_NC = 2
_NS = 16
_NW = _NC * _NS
_CH = 128

_SC_PARAMS = pltpu.CompilerParams(use_tc_tiling_on_sc=False)


def _rnd(n, m):
    return ((n + m - 1) // m) * m


def _bm(m, cap=2048):
    b = cap
    while m % b:
        b //= 2
    return b


@functools.lru_cache(maxsize=None)
def _make_gather(npad, d, epad):
    b = epad // _NW
    nch = b // _CH
    mesh = plsc.VectorSubcoreMesh(core_axis_name="c", subcore_axis_name="s")

    @functools.partial(
        pl.kernel,
        out_type=jax.ShapeDtypeStruct((epad, d), F32),
        mesh=mesh,
        compiler_params=_SC_PARAMS,
        scratch_types=[
            pltpu.VMEM((b,), jnp.int32),
            pltpu.VMEM((b, d), F32),
            pltpu.SemaphoreType.DMA,
        ],
    )
    def gather_k(table_hbm, idx_hbm, out_hbm, idx_v, rows_v, sem):
        wid = lax.axis_index("s") * _NC + lax.axis_index("c")
        pltpu.sync_copy(idx_hbm.at[pl.ds(wid * b, b)], idx_v)
        cps = []
        for j in range(nch):
            cps.append(pltpu.async_copy(
                table_hbm.at[idx_v.at[pl.ds(j * _CH, _CH)]],
                rows_v.at[pl.ds(j * _CH, _CH)], sem))
        for cp in cps:
            cp.wait()
        pltpu.sync_copy(rows_v, out_hbm.at[pl.ds(wid * b, b)])

    return gather_k


@functools.lru_cache(maxsize=None)
def _make_scatter(npad, d, epad, with_cnt=False):
    b = epad // _NW
    nch = b // _CH
    zr = npad // _NS
    mesh = plsc.VectorSubcoreMesh(core_axis_name="c", subcore_axis_name="s")

    out_type = [jax.ShapeDtypeStruct((_NC * npad, d), F32)]
    scratch = [
        pltpu.VMEM((b, d), F32),
        pltpu.VMEM((nch, _CH), jnp.int32),
        pltpu.VMEM_SHARED((npad, d), F32),
        pltpu.SemaphoreType.DMA,
    ]
    if with_cnt:
        out_type.append(jax.ShapeDtypeStruct((_NC * npad, 16), F32))
        scratch += [pltpu.VMEM((_CH, 16), F32),
                    pltpu.VMEM_SHARED((npad, 16), F32)]

    def scatter_body(vals_hbm, idx_hbm, zeros_hbm, *rest):
        if with_cnt:
            (zeros_c_hbm, ones_hbm, out_hbm, outc_hbm,
             vals_v, idx_v, acc_sh, sem, ones_v, accc_sh) = rest
        else:
            out_hbm, vals_v, idx_v, acc_sh, sem = rest
        c = lax.axis_index("c")
        s = lax.axis_index("s")
        wid = s * _NC + c
        pltpu.sync_copy(zeros_hbm.at[pl.ds(s * zr, zr)],
                        acc_sh.at[pl.ds(s * zr, zr)])
        if with_cnt:
            pltpu.sync_copy(zeros_c_hbm.at[pl.ds(s * zr, zr)],
                            accc_sh.at[pl.ds(s * zr, zr)])
            pltpu.sync_copy(ones_hbm, ones_v)
        plsc.subcore_barrier()
        pltpu.sync_copy(vals_hbm.at[pl.ds(wid * b, b)], vals_v)
        pltpu.sync_copy(idx_hbm.at[wid], idx_v)
        for j in range(nch):
            pltpu.sync_copy(vals_v.at[pl.ds(j * _CH, _CH)],
                            acc_sh.at[idx_v.at[j]], add=True)
            if with_cnt:
                pltpu.sync_copy(ones_v, accc_sh.at[idx_v.at[j]], add=True)
        plsc.subcore_barrier()
        pltpu.sync_copy(acc_sh.at[pl.ds(s * zr, zr)],
                        out_hbm.at[pl.ds(c * npad + s * zr, zr)])
        if with_cnt:
            pltpu.sync_copy(accc_sh.at[pl.ds(s * zr, zr)],
                            outc_hbm.at[pl.ds(c * npad + s * zr, zr)])

    return pl.kernel(
        scatter_body,
        out_type=out_type if with_cnt else out_type[0],
        mesh=mesh,
        compiler_params=_SC_PARAMS,
        scratch_types=scratch,
    )


def _mm(x, wt, bias, act=None):
    m, k = x.shape
    n = wt.shape[1]
    bm = _bm(m)

    def body(x_ref, w_ref, b_ref, o_ref):
        y = jnp.dot(x_ref[...], w_ref[...], preferred_element_type=F32) + b_ref[...]
        if act == 'relu':
            y = jnp.maximum(y, 0.0)
        o_ref[...] = y

    return pl.pallas_call(
        body,
        grid=(m // bm,),
        in_specs=[pl.BlockSpec((bm, k), lambda i: (i, 0)),
                  pl.BlockSpec((k, n), lambda i: (0, 0)),
                  pl.BlockSpec((1, n), lambda i: (0, 0))],
        out_specs=pl.BlockSpec((bm, n), lambda i: (i, 0)),
        out_shape=jax.ShapeDtypeStruct((m, n), F32),
    )(x, wt, bias)


def _msg(hsrc, g1, a_mat, r_mat, w, kk1, bm=512):
    e = hsrc.shape[0]
    n = a_mat.shape[1]
    bf16 = jnp.bfloat16

    def body(h_ref, g_ref, a_ref, r_ref, o_ref):
        p = jnp.dot(h_ref[...].astype(bf16), a_ref[...],
                    preferred_element_type=F32)
        t = jnp.dot(g_ref[...].astype(bf16), r_ref[...],
                    preferred_element_type=F32)
        q = p * t
        acc = q[:, :w]
        for k in range(1, kk1):
            acc = acc + q[:, k * w:(k + 1) * w]
        o_ref[...] = acc

    return pl.pallas_call(
        body,
        grid=(e // bm,),
        in_specs=[pl.BlockSpec((bm, hsrc.shape[1]), lambda i: (i, 0)),
                  pl.BlockSpec((bm, g1.shape[1]), lambda i: (i, 0)),
                  pl.BlockSpec((a_mat.shape[0], n), lambda i: (0, 0)),
                  pl.BlockSpec((r_mat.shape[0], n), lambda i: (0, 0))],
        out_specs=pl.BlockSpec((bm, w), lambda i: (i, 0)),
        out_shape=jax.ShapeDtypeStruct((e, w), F32),
    )(hsrc, g1, a_mat, r_mat)


def _gru(acc, cnt, hprev, wit, wht, bi, bh):
    npad, d = hprev.shape
    bm = _bm(npad)
    nb = npad // bm

    def body(a0_ref, a1_ref, c0_ref, c1_ref, h_ref, wi_ref, wh_ref,
             bi_ref, bh_ref, o_ref):
        s = a0_ref[...] + a1_ref[...]
        c = jnp.maximum(c0_ref[:, :1] + c1_ref[:, :1], 1.0)
        m = jnp.maximum(s / c, 0.0)
        hh = h_ref[...]
        gi = jnp.dot(m, wi_ref[...], preferred_element_type=F32) + bi_ref[...]
        gh = jnp.dot(hh, wh_ref[...], preferred_element_type=F32) + bh_ref[...]
        r = jax.nn.sigmoid(gi[:, :d] + gh[:, :d])
        z = jax.nn.sigmoid(gi[:, d:2 * d] + gh[:, d:2 * d])
        nn = jnp.tanh(gi[:, 2 * d:] + r * gh[:, 2 * d:])
        o_ref[...] = (1.0 - z) * nn + z * hh

    return pl.pallas_call(
        body,
        grid=(nb,),
        in_specs=[pl.BlockSpec((bm, d), lambda i: (i, 0)),
                  pl.BlockSpec((bm, d), lambda i, nb=nb: (i + nb, 0)),
                  pl.BlockSpec((bm, 16), lambda i: (i, 0)),
                  pl.BlockSpec((bm, 16), lambda i, nb=nb: (i + nb, 0)),
                  pl.BlockSpec((bm, d), lambda i: (i, 0)),
                  pl.BlockSpec((d, 3 * d), lambda i: (0, 0)),
                  pl.BlockSpec((d, 3 * d), lambda i: (0, 0)),
                  pl.BlockSpec((1, 3 * d), lambda i: (0, 0)),
                  pl.BlockSpec((1, 3 * d), lambda i: (0, 0))],
        out_specs=pl.BlockSpec((bm, d), lambda i: (i, 0)),
        out_shape=jax.ShapeDtypeStruct((npad, d), F32),
    )(acc, acc, cnt, cnt, hprev, wit, wht, bi, bh)


def _frag_assemble(ff, wt, bias, acc, cnt):
    npad = ff.shape[0]
    k = ff.shape[1]
    d = wt.shape[1]
    bm = _bm(npad)
    nb = npad // bm

    def body(f_ref, w_ref, b_ref, a0_ref, a1_ref, c0_ref, c1_ref, o_ref):
        emb = jnp.dot(f_ref[...], w_ref[...], preferred_element_type=F32) + b_ref[...]
        s = a0_ref[...] + a1_ref[...]
        c = jnp.maximum(c0_ref[:, :1] + c1_ref[:, :1], 1.0)
        o_ref[...] = jnp.concatenate([emb, s / c], axis=1)

    return pl.pallas_call(
        body,
        grid=(nb,),
        in_specs=[pl.BlockSpec((bm, k), lambda i: (i, 0)),
                  pl.BlockSpec((k, d), lambda i: (0, 0)),
                  pl.BlockSpec((1, d), lambda i: (0, 0)),
                  pl.BlockSpec((bm, d), lambda i: (i, 0)),
                  pl.BlockSpec((bm, d), lambda i, nb=nb: (i + nb, 0)),
                  pl.BlockSpec((bm, 16), lambda i: (i, 0)),
                  pl.BlockSpec((bm, 16), lambda i, nb=nb: (i + nb, 0))],
        out_specs=pl.BlockSpec((bm, 2 * d), lambda i: (i, 0)),
        out_shape=jax.ShapeDtypeStruct((npad, 2 * d), F32),
    )(ff, wt, bias, acc, acc, cnt, cnt)


def _final(acc, cnt, wt, bias, eps, nb_real, latent):
    npad = acc.shape[0] // 2
    d = acc.shape[1]

    def body(a0_ref, a1_ref, c0_ref, c1_ref, w_ref, b_ref, e_ref,
             z_ref, mu_ref, lv_ref):
        s = a0_ref[...] + a1_ref[...]
        c = jnp.maximum(c0_ref[:, :1] + c1_ref[:, :1], 1.0)
        hm = (s / c)[:nb_real]
        x = jnp.dot(hm, w_ref[...], preferred_element_type=F32) + b_ref[...]
        mu = x[:, :latent]
        lv = x[:, latent:]
        std = jnp.exp(0.5 * lv)
        z_ref[...] = mu + e_ref[...] * std
        mu_ref[...] = mu
        lv_ref[...] = lv

    out = jax.ShapeDtypeStruct((nb_real, latent), F32)
    return pl.pallas_call(
        body,
        grid=(1,),
        in_specs=[pl.BlockSpec((npad, d), lambda i: (0, 0)),
                  pl.BlockSpec((npad, d), lambda i: (1, 0)),
                  pl.BlockSpec((npad, 16), lambda i: (0, 0)),
                  pl.BlockSpec((npad, 16), lambda i: (1, 0)),
                  pl.BlockSpec((d, 2 * latent), lambda i: (0, 0)),
                  pl.BlockSpec((1, 2 * latent), lambda i: (0, 0)),
                  pl.BlockSpec((nb_real, latent), lambda i: (0, 0))],
        out_specs=[pl.BlockSpec((nb_real, latent), lambda i: (0, 0))] * 3,
        out_shape=[out, out, out],
    )(acc, acc, cnt, cnt, wt, bias, eps)


def _edge_net_mat(e2w, e2b, h, k):
    a = e2w.reshape(h, h, k).transpose(0, 2, 1).reshape(h, k * h)
    return jnp.concatenate([a, e2b.reshape(h, h)], axis=1)


def _pad_idx(idx, epad, dump):
    return jnp.pad(idx, (0, epad - idx.shape[0]), constant_values=dump)


def _pad_idx3(idx, epad, dump):
    return jnp.pad(idx, (0, epad - idx.shape[0]),
                   constant_values=dump).reshape(_NW, -1, _CH)


def kernel(atom_feat, atom_bond_feat, frag_feat, fbond_feat, atom_edge_index,
           atom_graph_ids, frag_edge_index, frag_graph_ids, eps, params):
    p = params
    na, ea = atom_feat.shape[0], atom_edge_index.shape[1]
    nf, ef = frag_feat.shape[0], frag_edge_index.shape[1]
    nb = eps.shape[0]
    latent = eps.shape[1]
    ha = p['emb_atom_W'].shape[0]
    hf = 2 * p['emb_frag_W'].shape[0]
    ka = p['amp']['e1_W'].shape[0]
    kf = p['fmp']['e1_W'].shape[0]

    nap = _rnd(na + 1, 1024)
    nfp = _rnd(nf + 1, 1024)
    nbp = _rnd(nb + 1, 128)
    eap = _rnd(ea, _NW * _CH)
    efp = _rnd(ef, _NW * _CH)
    iap = _rnd(max(na, nap), _NW * _CH)
    ifp = _rnd(max(nf, nfp), _NW * _CH)

    a_src = _pad_idx(atom_edge_index[0], eap, nap - 1)
    a_dst = _pad_idx3(atom_edge_index[1], eap, nap - 1)
    f_src = _pad_idx(frag_edge_index[0], efp, nfp - 1)
    f_dst = _pad_idx3(frag_edge_index[1], efp, nfp - 1)
    a_gid = _pad_idx3(atom_graph_ids, iap, nfp - 1)
    f_gid = _pad_idx3(frag_graph_ids, ifp, nbp - 1)

    amp, fmp = p['amp'], p['fmp']
    bf16 = jnp.bfloat16
    w_bond = (amp['e1_W'] @ p['emb_bond_W']).T
    b_bond = (p['emb_bond_b'] @ amp['e1_W'].T + amp['e1_b'])[None]
    w_fbond = (fmp['e1_W'] @ p['emb_fbond_W']).T
    b_fbond = (p['emb_fbond_b'] @ fmp['e1_W'].T + fmp['e1_b'])[None]
    w_bond = jnp.pad(w_bond, ((0, 0), (0, 1)))
    b_bond = jnp.concatenate([b_bond, jnp.ones((1, 1), F32)], axis=1)
    w_fbond = jnp.pad(w_fbond, ((0, 0), (0, 1)))
    b_fbond = jnp.concatenate([b_fbond, jnp.ones((1, 1), F32)], axis=1)
    eye4 = jnp.eye(4, dtype=F32)

    def lift_a(mat, hdim, kk1):
        m3 = mat.reshape(hdim, kk1, hdim)
        out = jnp.einsum('hko,jl->jhklo', m3, eye4)
        return out.reshape(4 * hdim, kk1 * 4 * hdim).astype(bf16)

    def lift_r(hdim, kk1):
        out = jnp.einsum('ak,jl,o->jaklo', jnp.eye(kk1, dtype=F32), eye4,
                         jnp.ones((hdim,), F32))
        return out.reshape(4 * kk1, kk1 * 4 * hdim).astype(bf16)

    a_mat_a = lift_a(_edge_net_mat(amp['e2_W'], amp['e2_b'], ha, ka), ha, ka + 1)
    a_mat_f = lift_a(_edge_net_mat(fmp['e2_W'], fmp['e2_b'], hf, kf), hf, kf + 1)
    r_a = lift_r(ha, ka + 1)
    r_f = lift_r(hf, kf + 1)
    w_bond4 = jnp.kron(eye4, w_bond)
    b_bond4 = jnp.tile(b_bond, (1, 4))
    w_fbond4 = jnp.kron(eye4, w_fbond)
    b_fbond4 = jnp.tile(b_fbond, (1, 4))

    zeros_a = jnp.zeros((nap, ha), F32)
    zeros_f32 = jnp.zeros((nfp, ha), F32)
    zeros_f64 = jnp.zeros((nfp, hf), F32)
    zeros_b = jnp.zeros((nbp, hf), F32)
    zeros_ca = jnp.zeros((nap, 16), F32)
    zeros_cf = jnp.zeros((nfp, 16), F32)
    zeros_cb = jnp.zeros((nbp, 16), F32)

    cnt_a = _make_scatter(nap, 16, eap)(jnp.ones((eap, 16), F32), a_dst, zeros_ca)
    cnt_f = _make_scatter(nfp, 16, efp)(jnp.ones((efp, 16), F32), f_dst, zeros_cf)
    cnt_af = _make_scatter(nfp, 16, iap)(jnp.ones((iap, 16), F32), a_gid, zeros_cf)
    cnt_fb = _make_scatter(nbp, 16, ifp)(jnp.ones((ifp, 16), F32), f_gid, zeros_cb)

    af = jnp.pad(atom_feat, ((0, nap - na), (0, 0)))
    h = _mm(af, p['emb_atom_W'].T, p['emb_atom_b'][None])
    bf4 = jnp.pad(atom_bond_feat, ((0, eap - ea), (0, 0))).reshape(eap // 4, -1)
    g_a = _mm(bf4, w_bond4, b_bond4, act='relu')

    gather_a = _make_gather(nap, ha, eap)
    wit_a, wht_a = amp['gru_Wih'].T, amp['gru_Whh'].T
    bi_a, bh_a = amp['gru_bih'][None], amp['gru_bhh'][None]
    for it in range(2):
        hs4 = gather_a(h, a_src).reshape(eap // 4, 4 * ha)
        msg4 = _msg(hs4, g_a, a_mat_a, r_a, 4 * ha, ka + 1)
        acc = _make_scatter(nap, ha, eap)(msg4.reshape(eap, ha), a_dst, zeros_a)
        h = _gru(acc, cnt_a, h, wit_a, wht_a, bi_a, bh_a)

    h_pad = jnp.pad(h, ((0, iap - nap), (0, 0)))
    acc_af = _make_scatter(nfp, ha, iap)(h_pad, a_gid, zeros_f32)
    ffp = jnp.pad(frag_feat, ((0, nfp - nf), (0, 0)))
    hfr = _frag_assemble(ffp, p['emb_frag_W'].T, p['emb_frag_b'][None],
                         acc_af, cnt_af)

    fbf4 = jnp.pad(fbond_feat, ((0, efp - ef), (0, 0))).reshape(efp // 4, -1)
    g_f = _mm(fbf4, w_fbond4, b_fbond4, act='relu')
    gather_f = _make_gather(nfp, hf, efp)
    wit_f, wht_f = fmp['gru_Wih'].T, fmp['gru_Whh'].T
    bi_f, bh_f = fmp['gru_bih'][None], fmp['gru_bhh'][None]
    for it in range(2):
        hs4 = gather_f(hfr, f_src).reshape(efp // 4, 4 * hf)
        msg4 = _msg(hs4, g_f, a_mat_f, r_f, 4 * hf, kf + 1)
        acc = _make_scatter(nfp, hf, efp)(msg4.reshape(efp, hf), f_dst, zeros_f64)
        hfr = _gru(acc, cnt_f, hfr, wit_f, wht_f, bi_f, bh_f)

    hfr_pad = jnp.pad(hfr, ((0, ifp - nfp), (0, 0)))
    acc_fb = _make_scatter(nbp, hf, ifp)(hfr_pad, f_gid, zeros_b)
    z, mu, lv = _final(acc_fb, cnt_fb, p['enc_W'].T, p['enc_b'][None],
                       eps, nb, latent)
    return (z, mu, lv)


# speedup vs baseline: 1.5713x; 1.1709x over previous
import functools

import jax
import jax.numpy as jnp
from jax import lax
from jax.experimental import pallas as pl
from jax.experimental.pallas import tpu as pltpu
from jax.experimental.pallas import tpu_sc as plsc

F32 = jnp.float32
_NC = 2
_NS = 16
_NW = _NC * _NS
_CH = 128

_SC_PARAMS = pltpu.CompilerParams(use_tc_tiling_on_sc=False)


def _rnd(n, m):
    return ((n + m - 1) // m) * m


def _bm(m, cap=2048):
    b = cap
    while m % b:
        b //= 2
    return b


@functools.lru_cache(maxsize=None)
def _make_gather(npad, d, epad):
    b = epad // _NW
    nch = b // _CH
    mesh = plsc.VectorSubcoreMesh(core_axis_name="c", subcore_axis_name="s")

    @functools.partial(
        pl.kernel,
        out_type=jax.ShapeDtypeStruct((epad, d), F32),
        mesh=mesh,
        compiler_params=_SC_PARAMS,
        scratch_types=[
            pltpu.VMEM((b,), jnp.int32),
            pltpu.VMEM((b, d), F32),
            pltpu.SemaphoreType.DMA,
        ],
    )
    def gather_k(table_hbm, idx_hbm, out_hbm, idx_v, rows_v, sem):
        wid = lax.axis_index("s") * _NC + lax.axis_index("c")
        pltpu.sync_copy(idx_hbm.at[pl.ds(wid * b, b)], idx_v)
        cps = []
        for j in range(nch):
            cps.append(pltpu.async_copy(
                table_hbm.at[idx_v.at[pl.ds(j * _CH, _CH)]],
                rows_v.at[pl.ds(j * _CH, _CH)], sem))
        for cp in cps:
            cp.wait()
        pltpu.sync_copy(rows_v, out_hbm.at[pl.ds(wid * b, b)])

    return gather_k


@functools.lru_cache(maxsize=None)
def _make_scatter(npad, d, epad, with_cnt=False):
    b = epad // _NW
    nch = b // _CH
    zr = npad // _NS
    mesh = plsc.VectorSubcoreMesh(core_axis_name="c", subcore_axis_name="s")

    out_type = [jax.ShapeDtypeStruct((_NC * npad, d), F32)]
    scratch = [
        pltpu.VMEM((b, d), F32),
        pltpu.VMEM((nch, _CH), jnp.int32),
        pltpu.VMEM_SHARED((npad, d), F32),
        pltpu.SemaphoreType.DMA,
    ]
    if with_cnt:
        out_type.append(jax.ShapeDtypeStruct((_NC * npad, 16), F32))
        scratch += [pltpu.VMEM((_CH, 16), F32),
                    pltpu.VMEM_SHARED((npad, 16), F32)]

    def scatter_body(vals_hbm, idx_hbm, zeros_hbm, *rest):
        if with_cnt:
            (zeros_c_hbm, ones_hbm, out_hbm, outc_hbm,
             vals_v, idx_v, acc_sh, sem, ones_v, accc_sh) = rest
        else:
            out_hbm, vals_v, idx_v, acc_sh, sem = rest
        c = lax.axis_index("c")
        s = lax.axis_index("s")
        wid = s * _NC + c
        pltpu.sync_copy(zeros_hbm.at[pl.ds(s * zr, zr)],
                        acc_sh.at[pl.ds(s * zr, zr)])
        if with_cnt:
            pltpu.sync_copy(zeros_c_hbm.at[pl.ds(s * zr, zr)],
                            accc_sh.at[pl.ds(s * zr, zr)])
            pltpu.sync_copy(ones_hbm, ones_v)
        plsc.subcore_barrier()
        pltpu.sync_copy(vals_hbm.at[pl.ds(wid * b, b)], vals_v)
        pltpu.sync_copy(idx_hbm.at[wid], idx_v)
        for j in range(nch):
            pltpu.sync_copy(vals_v.at[pl.ds(j * _CH, _CH)],
                            acc_sh.at[idx_v.at[j]], add=True)
            if with_cnt:
                pltpu.sync_copy(ones_v, accc_sh.at[idx_v.at[j]], add=True)
        plsc.subcore_barrier()
        pltpu.sync_copy(acc_sh.at[pl.ds(s * zr, zr)],
                        out_hbm.at[pl.ds(c * npad + s * zr, zr)])
        if with_cnt:
            pltpu.sync_copy(accc_sh.at[pl.ds(s * zr, zr)],
                            outc_hbm.at[pl.ds(c * npad + s * zr, zr)])

    return pl.kernel(
        scatter_body,
        out_type=out_type if with_cnt else out_type[0],
        mesh=mesh,
        compiler_params=_SC_PARAMS,
        scratch_types=scratch,
    )


def _mm(x, wt, bias, act=None):
    m, k = x.shape
    n = wt.shape[1]
    bm = _bm(m)

    def body(x_ref, w_ref, b_ref, o_ref):
        y = jnp.dot(x_ref[...], w_ref[...], preferred_element_type=F32) + b_ref[...]
        if act == 'relu':
            y = jnp.maximum(y, 0.0)
        o_ref[...] = y

    return pl.pallas_call(
        body,
        grid=(m // bm,),
        in_specs=[pl.BlockSpec((bm, k), lambda i: (i, 0)),
                  pl.BlockSpec((k, n), lambda i: (0, 0)),
                  pl.BlockSpec((1, n), lambda i: (0, 0))],
        out_specs=pl.BlockSpec((bm, n), lambda i: (i, 0)),
        out_shape=jax.ShapeDtypeStruct((m, n), F32),
    )(x, wt, bias)


def _msg(hsrc, g1, a_mat, r_mat, w, kk1, bm=512):
    e = hsrc.shape[0]
    n = a_mat.shape[1]
    bf16 = jnp.bfloat16

    def body(h_ref, g_ref, a_ref, r_ref, o_ref):
        p = jnp.dot(h_ref[...].astype(bf16), a_ref[...],
                    preferred_element_type=F32)
        t = jnp.dot(g_ref[...].astype(bf16), r_ref[...],
                    preferred_element_type=F32)
        q = p * t
        acc = q[:, :w]
        for k in range(1, kk1):
            acc = acc + q[:, k * w:(k + 1) * w]
        o_ref[...] = acc

    return pl.pallas_call(
        body,
        grid=(e // bm,),
        in_specs=[pl.BlockSpec((bm, hsrc.shape[1]), lambda i: (i, 0)),
                  pl.BlockSpec((bm, g1.shape[1]), lambda i: (i, 0)),
                  pl.BlockSpec((a_mat.shape[0], n), lambda i: (0, 0)),
                  pl.BlockSpec((r_mat.shape[0], n), lambda i: (0, 0))],
        out_specs=pl.BlockSpec((bm, w), lambda i: (i, 0)),
        out_shape=jax.ShapeDtypeStruct((e, w), F32),
    )(hsrc, g1, a_mat, r_mat)


def _gru(acc4, cnt4, h4, wit4, wht4, bi4, bh4):
    rows, w = h4.shape
    bm = _bm(rows)
    nb = rows // bm

    def body(a0_ref, a1_ref, c0_ref, c1_ref, h_ref, wi_ref, wh_ref,
             bi_ref, bh_ref, o_ref):
        s = a0_ref[...] + a1_ref[...]
        c = jnp.maximum(c0_ref[...] + c1_ref[...], 1.0)
        m = jnp.maximum(s / c, 0.0)
        hh = h_ref[...]
        gi = jnp.dot(m, wi_ref[...], preferred_element_type=F32) + bi_ref[...]
        gh = jnp.dot(hh, wh_ref[...], preferred_element_type=F32) + bh_ref[...]
        r = jax.nn.sigmoid(gi[:, :w] + gh[:, :w])
        z = jax.nn.sigmoid(gi[:, w:2 * w] + gh[:, w:2 * w])
        nn = jnp.tanh(gi[:, 2 * w:] + r * gh[:, 2 * w:])
        o_ref[...] = (1.0 - z) * nn + z * hh

    return pl.pallas_call(
        body,
        grid=(nb,),
        in_specs=[pl.BlockSpec((bm, w), lambda i: (i, 0)),
                  pl.BlockSpec((bm, w), lambda i, nb=nb: (i + nb, 0)),
                  pl.BlockSpec((bm, w), lambda i: (i, 0)),
                  pl.BlockSpec((bm, w), lambda i, nb=nb: (i + nb, 0)),
                  pl.BlockSpec((bm, w), lambda i: (i, 0)),
                  pl.BlockSpec((w, 3 * w), lambda i: (0, 0)),
                  pl.BlockSpec((w, 3 * w), lambda i: (0, 0)),
                  pl.BlockSpec((1, 3 * w), lambda i: (0, 0)),
                  pl.BlockSpec((1, 3 * w), lambda i: (0, 0))],
        out_specs=pl.BlockSpec((bm, w), lambda i: (i, 0)),
        out_shape=jax.ShapeDtypeStruct((rows, w), F32),
    )(acc4, acc4, cnt4, cnt4, h4, wit4, wht4, bi4, bh4)


def _frag_assemble(ff4, wt4, bias4, pmat, acc4, cnt4):
    rows, k = ff4.shape
    wo = wt4.shape[1]
    bm = _bm(rows)
    nb = rows // bm

    def body(f_ref, w_ref, b_ref, p_ref, a0_ref, a1_ref, c0_ref, c1_ref, o_ref):
        emb = jnp.dot(f_ref[...], w_ref[...], preferred_element_type=F32) + b_ref[...]
        sc = a0_ref[...] + a1_ref[...]
        c = jnp.maximum(c0_ref[...] + c1_ref[...], 1.0)
        mean = sc / c
        o_ref[...] = emb + jnp.dot(mean, p_ref[...], preferred_element_type=F32)

    wa = acc4.shape[1]
    return pl.pallas_call(
        body,
        grid=(nb,),
        in_specs=[pl.BlockSpec((bm, k), lambda i: (i, 0)),
                  pl.BlockSpec((k, wo), lambda i: (0, 0)),
                  pl.BlockSpec((1, wo), lambda i: (0, 0)),
                  pl.BlockSpec((wa, wo), lambda i: (0, 0)),
                  pl.BlockSpec((bm, wa), lambda i: (i, 0)),
                  pl.BlockSpec((bm, wa), lambda i, nb=nb: (i + nb, 0)),
                  pl.BlockSpec((bm, wa), lambda i: (i, 0)),
                  pl.BlockSpec((bm, wa), lambda i, nb=nb: (i + nb, 0))],
        out_specs=pl.BlockSpec((bm, wo), lambda i: (i, 0)),
        out_shape=jax.ShapeDtypeStruct((rows, wo), F32),
    )(ff4, wt4, bias4, pmat, acc4, acc4, cnt4, cnt4)


def _final(acc, cnt, wt, bias, eps, nb_real, latent):
    npad = acc.shape[0] // 2
    d = acc.shape[1]

    def body(a0_ref, a1_ref, c0_ref, c1_ref, w_ref, b_ref, e_ref,
             z_ref, mu_ref, lv_ref):
        s = a0_ref[...] + a1_ref[...]
        c = jnp.maximum(c0_ref[:, :1] + c1_ref[:, :1], 1.0)
        hm = (s / c)[:nb_real]
        x = jnp.dot(hm, w_ref[...], preferred_element_type=F32) + b_ref[...]
        mu = x[:, :latent]
        lv = x[:, latent:]
        std = jnp.exp(0.5 * lv)
        z_ref[...] = mu + e_ref[...] * std
        mu_ref[...] = mu
        lv_ref[...] = lv

    out = jax.ShapeDtypeStruct((nb_real, latent), F32)
    return pl.pallas_call(
        body,
        grid=(1,),
        in_specs=[pl.BlockSpec((npad, d), lambda i: (0, 0)),
                  pl.BlockSpec((npad, d), lambda i: (1, 0)),
                  pl.BlockSpec((npad, 16), lambda i: (0, 0)),
                  pl.BlockSpec((npad, 16), lambda i: (1, 0)),
                  pl.BlockSpec((d, 2 * latent), lambda i: (0, 0)),
                  pl.BlockSpec((1, 2 * latent), lambda i: (0, 0)),
                  pl.BlockSpec((nb_real, latent), lambda i: (0, 0))],
        out_specs=[pl.BlockSpec((nb_real, latent), lambda i: (0, 0))] * 3,
        out_shape=[out, out, out],
    )(acc, acc, cnt, cnt, wt, bias, eps)


def _edge_net_mat(e2w, e2b, h, k):
    a = e2w.reshape(h, h, k).transpose(0, 2, 1).reshape(h, k * h)
    return jnp.concatenate([a, e2b.reshape(h, h)], axis=1)


def _pad_idx(idx, epad, dump):
    return jnp.pad(idx, (0, epad - idx.shape[0]), constant_values=dump)


def _pad_idx3(idx, epad, dump):
    return jnp.pad(idx, (0, epad - idx.shape[0]),
                   constant_values=dump).reshape(_NW, -1, _CH)


def kernel(atom_feat, atom_bond_feat, frag_feat, fbond_feat, atom_edge_index,
           atom_graph_ids, frag_edge_index, frag_graph_ids, eps, params):
    p = params
    na, ea = atom_feat.shape[0], atom_edge_index.shape[1]
    nf, ef = frag_feat.shape[0], frag_edge_index.shape[1]
    nb = eps.shape[0]
    latent = eps.shape[1]
    ha = p['emb_atom_W'].shape[0]
    hf = 2 * p['emb_frag_W'].shape[0]
    ka = p['amp']['e1_W'].shape[0]
    kf = p['fmp']['e1_W'].shape[0]

    nap = _rnd(na + 1, 1024)
    nfp = _rnd(nf + 1, 1024)
    nbp = _rnd(nb + 1, 128)
    eap = _rnd(ea, _NW * _CH)
    efp = _rnd(ef, _NW * _CH)
    iap = _rnd(max(na, nap), _NW * _CH)
    ifp = _rnd(max(nf, nfp), _NW * _CH)

    a_src = _pad_idx(atom_edge_index[0], eap, nap - 1)
    a_dst = _pad_idx3(atom_edge_index[1], eap, nap - 1)
    f_src = _pad_idx(frag_edge_index[0], efp, nfp - 1)
    f_dst = _pad_idx3(frag_edge_index[1], efp, nfp - 1)
    a_gid = _pad_idx3(atom_graph_ids, iap, nfp - 1)
    f_gid = _pad_idx3(frag_graph_ids, ifp, nbp - 1)

    amp, fmp = p['amp'], p['fmp']
    bf16 = jnp.bfloat16
    w_bond = (amp['e1_W'] @ p['emb_bond_W']).T
    b_bond = (p['emb_bond_b'] @ amp['e1_W'].T + amp['e1_b'])[None]
    w_fbond = (fmp['e1_W'] @ p['emb_fbond_W']).T
    b_fbond = (p['emb_fbond_b'] @ fmp['e1_W'].T + fmp['e1_b'])[None]
    w_bond = jnp.pad(w_bond, ((0, 0), (0, 1)))
    b_bond = jnp.concatenate([b_bond, jnp.ones((1, 1), F32)], axis=1)
    w_fbond = jnp.pad(w_fbond, ((0, 0), (0, 1)))
    b_fbond = jnp.concatenate([b_fbond, jnp.ones((1, 1), F32)], axis=1)
    eye4 = jnp.eye(4, dtype=F32)

    def lift_a(mat, hdim, kk1):
        m3 = mat.reshape(hdim, kk1, hdim)
        out = jnp.einsum('hko,jl->jhklo', m3, eye4)
        return out.reshape(4 * hdim, kk1 * 4 * hdim).astype(bf16)

    def lift_r(hdim, kk1):
        out = jnp.einsum('ak,jl,o->jaklo', jnp.eye(kk1, dtype=F32), eye4,
                         jnp.ones((hdim,), F32))
        return out.reshape(4 * kk1, kk1 * 4 * hdim).astype(bf16)

    a_mat_a = lift_a(_edge_net_mat(amp['e2_W'], amp['e2_b'], ha, ka), ha, ka + 1)
    a_mat_f = lift_a(_edge_net_mat(fmp['e2_W'], fmp['e2_b'], hf, kf), hf, kf + 1)
    r_a = lift_r(ha, ka + 1)
    r_f = lift_r(hf, kf + 1)

    def lift_gru_w(wt, d):
        w3 = wt.reshape(d, 3, d)
        return jnp.einsum('fgc,jl->jfglc', w3, eye4).reshape(4 * d, 12 * d)

    def lift_gru_b(b, d):
        return jnp.broadcast_to(b.reshape(3, 1, d), (3, 4, d)).reshape(1, 12 * d)
    w_bond4 = jnp.kron(eye4, w_bond)
    b_bond4 = jnp.tile(b_bond, (1, 4))
    w_fbond4 = jnp.kron(eye4, w_fbond)
    b_fbond4 = jnp.tile(b_fbond, (1, 4))

    zeros_a = jnp.zeros((nap, ha), F32)
    zeros_f32 = jnp.zeros((nfp, ha), F32)
    zeros_f64 = jnp.zeros((nfp, hf), F32)
    zeros_b = jnp.zeros((nbp, hf), F32)
    zeros_ca = jnp.zeros((nap, 16), F32)
    zeros_cf = jnp.zeros((nfp, 16), F32)
    zeros_cb = jnp.zeros((nbp, 16), F32)

    cnt_a = _make_scatter(nap, ha, eap)(jnp.ones((eap, ha), F32), a_dst, zeros_a)
    cnt_f = _make_scatter(nfp, hf, efp)(jnp.ones((efp, hf), F32), f_dst, zeros_f64)
    cnt_af = _make_scatter(nfp, ha, iap)(jnp.ones((iap, ha), F32), a_gid, zeros_f32)
    cnt_fb = _make_scatter(nbp, 16, ifp)(jnp.ones((ifp, 16), F32), f_gid, zeros_cb)
    cnt_a4 = cnt_a.reshape(2 * nap // 4, 4 * ha)
    cnt_f4 = cnt_f.reshape(2 * nfp // 4, 4 * hf)
    cnt_af4 = cnt_af.reshape(2 * nfp // 4, 4 * ha)

    af4 = jnp.pad(atom_feat, ((0, nap - na), (0, 0))).reshape(nap // 4, -1)
    h4 = _mm(af4, jnp.kron(eye4, p['emb_atom_W'].T),
             jnp.tile(p['emb_atom_b'][None], (1, 4)))
    bf4 = jnp.pad(atom_bond_feat.reshape(ea // 4, -1),
                  ((0, (eap - ea) // 4), (0, 0)))
    g_a = _mm(bf4, w_bond4, b_bond4, act='relu')

    gather_a = _make_gather(nap, ha, eap)
    wit_a = lift_gru_w(amp['gru_Wih'].T, ha)
    wht_a = lift_gru_w(amp['gru_Whh'].T, ha)
    bi_a = lift_gru_b(amp['gru_bih'], ha)
    bh_a = lift_gru_b(amp['gru_bhh'], ha)
    for it in range(2):
        hs4 = gather_a(h4.reshape(nap, ha), a_src).reshape(eap // 4, 4 * ha)
        msg4 = _msg(hs4, g_a, a_mat_a, r_a, 4 * ha, ka + 1)
        acc = _make_scatter(nap, ha, eap)(msg4.reshape(eap, ha), a_dst, zeros_a)
        h4 = _gru(acc.reshape(2 * nap // 4, 4 * ha), cnt_a4, h4,
                  wit_a, wht_a, bi_a, bh_a)

    h_pad = jnp.pad(h4, ((0, (iap - nap) // 4), (0, 0))).reshape(iap, ha)
    acc_af = _make_scatter(nfp, ha, iap)(h_pad, a_gid, zeros_f32)
    ff4 = jnp.pad(frag_feat, ((0, nfp - nf), (0, 0))).reshape(nfp // 4, -1)
    emb_w4 = jnp.einsum('fc,jl->jflc',
                        jnp.pad(p['emb_frag_W'].T, ((0, 0), (0, ha))),
                        eye4).reshape(4 * frag_feat.shape[1], 4 * hf)
    emb_b4 = jnp.tile(jnp.pad(p['emb_frag_b'], (0, ha))[None], (1, 4))
    pmat = jnp.kron(eye4, jnp.pad(jnp.eye(ha, dtype=F32), ((0, 0), (ha, 0))))
    hfr4 = _frag_assemble(ff4, emb_w4, emb_b4, pmat,
                          acc_af.reshape(2 * nfp // 4, 4 * ha), cnt_af4)

    fbf4 = jnp.pad(fbond_feat.reshape(ef // 4, -1),
                   ((0, (efp - ef) // 4), (0, 0)))
    g_f = _mm(fbf4, w_fbond4, b_fbond4, act='relu')
    gather_f = _make_gather(nfp, hf, efp)
    wit_f = lift_gru_w(fmp['gru_Wih'].T, hf)
    wht_f = lift_gru_w(fmp['gru_Whh'].T, hf)
    bi_f = lift_gru_b(fmp['gru_bih'], hf)
    bh_f = lift_gru_b(fmp['gru_bhh'], hf)
    for it in range(2):
        hs4 = gather_f(hfr4.reshape(nfp, hf), f_src).reshape(efp // 4, 4 * hf)
        msg4 = _msg(hs4, g_f, a_mat_f, r_f, 4 * hf, kf + 1)
        acc = _make_scatter(nfp, hf, efp)(msg4.reshape(efp, hf), f_dst, zeros_f64)
        hfr4 = _gru(acc.reshape(2 * nfp // 4, 4 * hf), cnt_f4, hfr4,
                    wit_f, wht_f, bi_f, bh_f)

    hfr_pad = jnp.pad(hfr4, ((0, (ifp - nfp) // 4), (0, 0))).reshape(ifp, hf)
    acc_fb = _make_scatter(nbp, hf, ifp)(hfr_pad, f_gid, zeros_b)
    z, mu, lv = _final(acc_fb, cnt_fb, p['enc_W'].T, p['enc_b'][None],
                       eps, nb, latent)
    return (z, mu, lv)
